# baseline (speedup 1.0000x reference)
import jax
import jax.numpy as jnp
from jax import lax
from jax.experimental import pallas as pl
from jax.experimental.pallas import tpu as pltpu, tpu_sc as plsc

VOCAB = 100000
EMBED_DIM = 64
BATCH = 16384
SEQ = 200
N_TOKENS = 20
SPLIT1 = 10

NC, NS, L = 2, 16, 16
NW = NC * NS
TOTAL = BATCH * SEQ
C = 800
ROWS_PER_W = TOTAL // NW
N_CHUNKS = ROWS_PER_W // C
P = 400
SUBS = (128, 128, 128, 128, 128, 128, 32)


def _body(tok_hbm, table_hbm, out_hbm,
          tok_v0, tok_v1, idx_v0, idx_v1, rows_v0, rows_v1, lv_t,
          stok0, stok1, sg0, sg1, sout0, sout1):
    wid = lax.axis_index("s") * NC + lax.axis_index("c")
    iota = lax.iota(jnp.int32, L)
    tok_v = (tok_v0, tok_v1)
    idx_v = (idx_v0, idx_v1)
    rows_v = (rows_v0, rows_v1)
    stok = (stok0, stok1)
    sg = (sg0, sg1)
    sout = (sout0, sout1)

    for g in range(P // L):
        j = (g * L + iota) % SEQ
        lv = jnp.where(
            (j >= 1) & (j <= SPLIT1), VOCAB + j - 1,
            jnp.where((j >= SPLIT1 + 2) & (j <= N_TOKENS + 1), VOCAB + j - 2,
                      -1))
        lv_t[pl.ds(g * L, L)] = lv

    def fire_tok(i, b):
        r0 = wid * ROWS_PER_W + i * C
        return pltpu.async_copy(tok_hbm.at[pl.ds(r0, C)], tok_v[b], stok[b])

    def build_idx(b):
        pltpu.make_async_copy(tok_hbm.at[pl.ds(0, C)], tok_v[b], stok[b]).wait()
        for g in range(C // L):
            t = (g * L) % P
            lv = lv_t[pl.ds(t, L)]
            idx = jnp.where(lv >= 0, lv, tok_v[b][pl.ds(g * L, L)])
            if any((g * L + l) % SEQ == SPLIT1 + 1 for l in range(L)):
                shifted = tok_v[b][pl.ds(g * L + (N_TOKENS - SPLIT1), L)]
                jvec = (g * L + iota) % SEQ
                idx = jnp.where(jvec == SPLIT1 + 1, shifted, idx)
            idx_v[b][pl.ds(g * L, L)] = idx

    def gather_parts(b):
        for m in range(C // SEQ):
            for off, nsub in ((0, 128), (128, SEQ - 128)):
                yield (table_hbm.at[idx_v[b].at[pl.ds(m * SEQ + off, nsub)]],
                       rows_v[b].at[m, pl.ds(off, nsub)])

    def fire_gather(b):
        for src, dst in gather_parts(b):
            pltpu.async_copy(src, dst, sg[b])

    def wait_gather(b):
        for src, dst in gather_parts(b):
            pltpu.make_async_copy(src, dst, sg[b]).wait()

    def fire_out(i, b):
        b0 = wid * (ROWS_PER_W // SEQ) + i * (C // SEQ)
        pltpu.async_copy(rows_v[b], out_hbm.at[pl.ds(b0, C // SEQ)], sout[b])

    def wait_out(b):
        pltpu.make_async_copy(
            rows_v[b], out_hbm.at[pl.ds(0, C // SEQ)], sout[b]).wait()

    fire_tok(0, 0)
    build_idx(0)
    fire_gather(0)
    fire_tok(1, 1)
    build_idx(1)
    wait_gather(0)
    fire_out(0, 0)
    fire_gather(1)
    fire_tok(2, 0)

    def steady(i, carry):
        b = lax.rem(i, 2)

        def one(b, bo):
            build_idx(b)
            wait_gather(bo)
            fire_out(i - 1, bo)
            wait_out(b)
            fire_gather(b)
            fire_tok(i + 1, bo)

        lax.cond(b == 0, lambda: one(0, 1), lambda: one(1, 0))
        return carry

    lax.fori_loop(2, N_CHUNKS - 1, steady, 0)

    bl = (N_CHUNKS - 1) % 2
    build_idx(bl)
    wait_gather(1 - bl)
    fire_out(N_CHUNKS - 2, 1 - bl)
    wait_out(bl)
    fire_gather(bl)
    wait_gather(bl)
    fire_out(N_CHUNKS - 1, bl)
    wait_out(1 - bl)
    wait_out(bl)


def kernel(tokens, wte_weight, learned_embedding):
    table = jnp.concatenate([wte_weight, learned_embedding], axis=0)
    tok_flat = tokens.reshape(TOTAL).astype(jnp.int32)
    mesh = plsc.VectorSubcoreMesh(core_axis_name="c", subcore_axis_name="s",
                                  num_cores=NC, num_subcores=NS)
    out = pl.kernel(
        _body,
        out_type=jax.ShapeDtypeStruct((BATCH, SEQ, EMBED_DIM), jnp.float32),
        mesh=mesh,
        compiler_params=pltpu.CompilerParams(use_tc_tiling_on_sc=False),
        scratch_types=[
            pltpu.VMEM((C,), jnp.int32),
            pltpu.VMEM((C,), jnp.int32),
            pltpu.VMEM((C,), jnp.int32),
            pltpu.VMEM((C,), jnp.int32),
            pltpu.VMEM((C // SEQ, SEQ, EMBED_DIM), jnp.float32),
            pltpu.VMEM((C // SEQ, SEQ, EMBED_DIM), jnp.float32),
            pltpu.VMEM((P,), jnp.int32),
            pltpu.SemaphoreType.DMA,
            pltpu.SemaphoreType.DMA,
            pltpu.SemaphoreType.DMA,
            pltpu.SemaphoreType.DMA,
            pltpu.SemaphoreType.DMA,
            pltpu.SemaphoreType.DMA,
        ],
    )(tok_flat, table)
    return out
